# baseline (speedup 1.0000x reference)
import functools

import jax
import jax.numpy as jnp
from jax import lax
from jax.experimental import pallas as pl
from jax.experimental.pallas import tpu as pltpu
from jax.experimental.pallas import tpu_sc as plsc

B, C, H, W = 32, 64, 128, 128
HW = H * W
K = 500
KP = 512
NC, NS, L = 2, 16, 16
KB = 64
NBLK = KP // KB
BW = KB * C


def _sc_body(outs_hbm, tgt_hbm, cind_hbm, cidx_hbm, nbs_hbm, out_hbm,
             cind_v, cidx_v, nb_v, idx0, idx1, pred0, pred1, tgt_all,
             res_v, sem0, sem1, tsem):
    b = lax.axis_index("s") * NC + lax.axis_index("c")
    tcopy = pltpu.make_async_copy(tgt_hbm.at[b], tgt_all.at[pl.ds(0, K), :],
                                  tsem)
    tcopy.start()
    pltpu.sync_copy(cind_hbm.at[b], cind_v)
    pltpu.sync_copy(cidx_hbm.at[b], cidx_v)
    pltpu.sync_copy(nbs_hbm.at[b], nb_v)
    nb = nb_v[pl.ds(0, L)][0]

    iota = lax.iota(jnp.int32, L)
    cvecs = [(iota + t * L) * HW + b * (C * HW) for t in range(C // L)]

    bufs = ((idx0, pred0, sem0), (idx1, pred1, sem1))

    def fire(blk, buf):
        idx_v, pred_v, sem = buf
        k0 = blk * KB

        @pl.when(k0 < nb)
        def _():
            for rc in range(KB // L):
                cv = cind_v[pl.ds(k0 + rc * L, L)]
                for i in range(L):
                    s = cv[i]
                    for t in range(C // L):
                        idx_v[pl.ds((rc * L + i) * C + t * L, L)] = (
                            cvecs[t] + s)
            pltpu.make_async_copy(outs_hbm.at[idx_v], pred_v, sem).start()

    def drain_accum(blk, buf, accs):
        idx_v, pred_v, sem = buf
        k0 = blk * KB

        @pl.when(k0 < nb)
        def _():
            pltpu.make_async_copy(outs_hbm.at[idx_v], pred_v, sem).wait()

        accs = list(accs)
        for rc in range(KB // L):
            kv = cidx_v[pl.ds(k0 + rc * L, L)]
            for i in range(L):
                r = rc * L + i
                kr = kv[i]
                ok = (k0 + r) < nb
                for t in range(C // L):
                    pv = pred_v[pl.ds(r * C + t * L, L)]
                    tv = tgt_all[kr, pl.ds(t * L, L)]
                    accs[t] = accs[t] + jnp.where(ok, jnp.abs(pv - tv), 0.0)
        return tuple(accs)

    zero = jnp.zeros((L,), jnp.float32)
    accs0 = (zero, zero, zero, zero)

    tcopy.wait()
    fire(0, bufs[0])

    def pair_step(i, accs):
        b0 = 2 * i
        fire(b0 + 1, bufs[1])
        accs = drain_accum(b0, bufs[0], accs)

        @pl.when(b0 + 2 < NBLK)
        def _():
            fire(b0 + 2, bufs[0])

        return drain_accum(b0 + 1, bufs[1], accs)

    accs = lax.fori_loop(0, NBLK // 2, pair_step, accs0)
    res_v[0, :] = accs[0] + accs[1] + accs[2] + accs[3]
    res_v[1, :] = jnp.where(iota == 0, nb.astype(jnp.float32), 0.0)
    pltpu.sync_copy(res_v, out_hbm.at[b])


@jax.jit
def kernel(outputs_key, targets_mask_key, targets_ind_key, targets_key):
    outs_flat = outputs_key.reshape(B * C * HW)

    mask = targets_mask_key
    pos = jnp.cumsum(mask, axis=1) - 1
    safe_pos = jnp.where(mask > 0, pos, KP - 1)
    rows = jnp.arange(B, dtype=jnp.int32)[:, None]
    karr = jnp.broadcast_to(jnp.arange(K, dtype=jnp.int32), (B, K))
    cidx = jnp.zeros((B, KP), jnp.int32).at[rows, safe_pos].set(karr)
    cind = jnp.take_along_axis(targets_ind_key, cidx, axis=1)
    nbs = jnp.broadcast_to(jnp.sum(mask, axis=1, dtype=jnp.int32)[:, None],
                           (B, L))

    mesh = plsc.VectorSubcoreMesh(core_axis_name="c", subcore_axis_name="s")
    f = pl.kernel(
        _sc_body,
        out_type=jax.ShapeDtypeStruct((B, 2, L), jnp.float32),
        mesh=mesh,
        scratch_types=[
            pltpu.VMEM((KP,), jnp.int32),
            pltpu.VMEM((KP,), jnp.int32),
            pltpu.VMEM((L,), jnp.int32),
            pltpu.VMEM((BW,), jnp.int32),
            pltpu.VMEM((BW,), jnp.int32),
            pltpu.VMEM((BW,), jnp.float32),
            pltpu.VMEM((BW,), jnp.float32),
            pltpu.VMEM((KP, C), jnp.float32),
            pltpu.VMEM((2, L), jnp.float32),
            pltpu.SemaphoreType.DMA,
            pltpu.SemaphoreType.DMA,
            pltpu.SemaphoreType.DMA,
        ],
    )
    part = f(outs_flat, targets_key, cind, cidx, nbs)
    num = jnp.sum(part[:, 0, :])
    cnt = jnp.sum(part[:, 1, :])
    loss = num / (B * K * C)
    return loss / (C * cnt + 0.0001)
